# baseline (speedup 1.0000x reference)
import functools
import math

import jax
import jax.numpy as jnp
from jax.experimental import pallas as pl

L = 64
K = 128
B = 4096
_HALF_LOG_2PI = 0.5 * math.log(2.0 * math.pi)


def _mog_block(z_ref, mT_ref, lvT_ref, w_ref, out_ref):
    zb = z_ref[...]
    mT = mT_ref[...]
    lvT = lvT_ref[...]
    wv = w_ref[...]

    wm = jnp.max(wv)
    logw = wv - (wm + jnp.log(jnp.sum(jnp.exp(wv - wm))))

    ha = 0.5 * jnp.exp(-lvT)
    c = (logw - _HALF_LOG_2PI) - 0.5 * lvT

    d = zb[:, :, None] - mT[None, :, :]
    x = c[None, :, :] - ha[None, :, :] * (d * d)
    m = jnp.max(x, axis=2)
    s = jnp.sum(jnp.exp(x - m[:, :, None]), axis=2)
    out_ref[...] = m + jnp.log(s)


@jax.jit
def kernel(z, means, logvars, w):
    mT = means.T
    lvT = logvars.T
    wrow = w.reshape(1, K)

    Bb = 256
    grid = (B // Bb,)
    out = pl.pallas_call(
        _mog_block,
        grid=grid,
        in_specs=[
            pl.BlockSpec((Bb, L), lambda i: (i, 0)),
            pl.BlockSpec((L, K), lambda i: (0, 0)),
            pl.BlockSpec((L, K), lambda i: (0, 0)),
            pl.BlockSpec((1, K), lambda i: (0, 0)),
        ],
        out_specs=pl.BlockSpec((Bb, L), lambda i: (i, 0)),
        out_shape=jax.ShapeDtypeStruct((B, L), jnp.float32),
    )(z, mT, lvT, wrow)
    return out.T


# speedup vs baseline: 1.8927x; 1.8927x over previous
import math

import jax
import jax.numpy as jnp
from jax.experimental import pallas as pl
from jax.experimental.pallas import tpu as pltpu

L = 64
K = 128
B = 4096
P = K // 2
_HALF_LOG_2PI = 0.5 * math.log(2.0 * math.pi)


def _mog_block(z_ref, mP_ref, lvP_ref, wP_ref, wrow_ref, out_ref, x_scr):
    Bb = z_ref.shape[0]
    zb = z_ref[...]
    zd = jnp.concatenate([zb, zb], axis=1)

    wrow = wrow_ref[...]
    wm = jnp.max(wrow)
    z_norm = wm + jnp.log(jnp.sum(jnp.exp(wrow - wm)))

    lvP = lvP_ref[...]
    haP = 0.5 * jnp.exp(-lvP)
    cP = (wP_ref[...] - z_norm - _HALF_LOG_2PI) - 0.5 * lvP

    m = jnp.full((Bb, 2 * L), -jnp.inf, dtype=jnp.float32)
    for p in range(P):
        d = zd - mP_ref[p:p + 1, :]
        x = cP[p:p + 1, :] - haP[p:p + 1, :] * (d * d)
        x_scr[p] = x
        m = jnp.maximum(m, x)

    s = jnp.zeros((Bb, 2 * L), dtype=jnp.float32)
    for p in range(P):
        s = s + jnp.exp(x_scr[p] - m)

    m0, m1 = m[:, :L], m[:, L:]
    s0, s1 = s[:, :L], s[:, L:]
    mt = jnp.maximum(m0, m1)
    st = s0 * jnp.exp(m0 - mt) + s1 * jnp.exp(m1 - mt)
    out_ref[...] = mt + jnp.log(st)


@jax.jit
def kernel(z, means, logvars, w):
    mP = means.reshape(P, 2 * L)
    lvP = logvars.reshape(P, 2 * L)
    wflat = w.reshape(K)
    wPp = jnp.broadcast_to(wflat.reshape(P, 2, 1), (P, 2, L)).reshape(P, 2 * L)
    wrow = wflat.reshape(1, K)

    Bb = 256
    grid = (B // Bb,)
    out = pl.pallas_call(
        _mog_block,
        grid=grid,
        in_specs=[
            pl.BlockSpec((Bb, L), lambda i: (i, 0)),
            pl.BlockSpec((P, 2 * L), lambda i: (0, 0)),
            pl.BlockSpec((P, 2 * L), lambda i: (0, 0)),
            pl.BlockSpec((P, 2 * L), lambda i: (0, 0)),
            pl.BlockSpec((1, K), lambda i: (0, 0)),
        ],
        out_specs=pl.BlockSpec((Bb, L), lambda i: (i, 0)),
        out_shape=jax.ShapeDtypeStruct((B, L), jnp.float32),
        scratch_shapes=[pltpu.VMEM((P, Bb, 2 * L), jnp.float32)],
    )(z, mP, lvP, wPp, wrow)
    return out.T
